# baseline (speedup 1.0000x reference)
import jax
import jax.numpy as jnp
from jax.experimental import pallas as pl
from jax.experimental.pallas import tpu as pltpu

ROW_BLK = 400
CLASSES = ((0, 6, 2432), (6, 6, 4864), (12, 6, 7296), (18, 7, 10000))


def _sweep1_kernel(adj_ref, x_ref, w1_ref, w23_ref,
                   tw_ref, mv_ref, xw1_s, tw_s):
    s = pl.program_id(0)
    r = pl.num_programs(0) - 1 - s

    @pl.when(s == 0)
    def _():
        xw1_s[...] = jnp.dot(
            x_ref[...], w1_ref[...], preferred_element_type=jnp.float32)
        tw_s[...] = jnp.zeros_like(tw_s)

    mv_ref[...] = jnp.dot(adj_ref[...], tw_s[...],
                          preferred_element_type=jnp.float32)

    temp = jnp.maximum(
        jnp.dot(adj_ref[...], xw1_s[...],
                preferred_element_type=jnp.float32), 0.0)
    twi = jnp.dot(temp, w23_ref[...], preferred_element_type=jnp.float32)
    tw_s[pl.ds(r * ROW_BLK, ROW_BLK), :] = twi
    tw_ref[...] = twi


def _decoder_kernel(z_ref, zt_ref, out_ref):
    out_ref[...] = jnp.dot(z_ref[...], zt_ref[...],
                           preferred_element_type=jnp.float32)


def kernel(node_vectors, adj, W1, W2, W3):
    n, d = node_vectors.shape
    h1 = W1.shape[1]
    h2 = W2.shape[1]
    w23 = jnp.concatenate([W2, W3], axis=1)

    nblk = n // ROW_BLK

    rev = lambda s: (nblk - 1 - s, 0)
    tw, mv = pl.pallas_call(
        _sweep1_kernel,
        grid=(nblk,),
        in_specs=[
            pl.BlockSpec((ROW_BLK, n), rev),
            pl.BlockSpec((n, d), lambda s: (0, 0)),
            pl.BlockSpec((d, h1), lambda s: (0, 0)),
            pl.BlockSpec((h1, 2 * h2), lambda s: (0, 0)),
        ],
        out_specs=[
            pl.BlockSpec((ROW_BLK, 2 * h2), rev),
            pl.BlockSpec((ROW_BLK, 2 * h2), rev),
        ],
        out_shape=[
            jax.ShapeDtypeStruct((n, 2 * h2), jnp.float32),
            jax.ShapeDtypeStruct((n, 2 * h2), jnp.float32),
        ],
        scratch_shapes=[
            pltpu.VMEM((n, h1), jnp.float32),
            pltpu.VMEM((n, 2 * h2), jnp.float32),
        ],
    )(adj, node_vectors, W1, w23)

    def make_class_kernel(off):
        def _class_kernel(adj_ref, tw_ref, mv_ref, mean_ref, logvar_ref):
            t = pl.program_id(0)
            limit = (t + off + 1) * ROW_BLK
            rows = jax.lax.broadcasted_iota(jnp.int32, tw_ref.shape, 0)
            twm = jnp.where(rows < limit, tw_ref[...], 0.0)
            acc = mv_ref[...] + jnp.dot(adj_ref[...], twm,
                                        preferred_element_type=jnp.float32)
            mean_ref[...] = acc[:, :h2]
            logvar_ref[...] = acc[:, h2:]
        return _class_kernel

    mean_parts, logvar_parts = [], []
    for off, cnt, width in CLASSES:
        m_c, lv_c = pl.pallas_call(
            make_class_kernel(off),
            grid=(cnt,),
            in_specs=[
                pl.BlockSpec((ROW_BLK, width), lambda t, off=off: (t + off, 0)),
                pl.BlockSpec((width, 2 * h2), lambda t: (0, 0)),
                pl.BlockSpec((ROW_BLK, 2 * h2), lambda t, off=off: (t + off, 0)),
            ],
            out_specs=[
                pl.BlockSpec((ROW_BLK, h2), lambda t: (t, 0)),
                pl.BlockSpec((ROW_BLK, h2), lambda t: (t, 0)),
            ],
            out_shape=[
                jax.ShapeDtypeStruct((cnt * ROW_BLK, h2), jnp.float32),
                jax.ShapeDtypeStruct((cnt * ROW_BLK, h2), jnp.float32),
            ],
        )(adj, tw, mv)
        mean_parts.append(m_c)
        logvar_parts.append(lv_c)

    mean = jnp.concatenate(mean_parts, axis=0)
    logvar = jnp.concatenate(logvar_parts, axis=0)

    adj_dec = pl.pallas_call(
        _decoder_kernel,
        grid=(nblk,),
        in_specs=[
            pl.BlockSpec((ROW_BLK, h2), lambda i: (i, 0)),
            pl.BlockSpec((h2, n), lambda i: (0, 0)),
        ],
        out_specs=pl.BlockSpec((ROW_BLK, n), lambda i: (i, 0)),
        out_shape=jax.ShapeDtypeStruct((n, n), jnp.float32),
    )(mean, mean.T)

    return (adj_dec, mean, logvar)


# speedup vs baseline: 1.0222x; 1.0222x over previous
import jax
import jax.numpy as jnp
from jax.experimental import pallas as pl
from jax.experimental.pallas import tpu as pltpu

ROW_BLK = 400


def _gcn_kernel(adj_ref, x_ref, w1_ref, w23_ref,
                mean_ref, logvar_ref,
                xw1_s, tw_s):
    p = pl.program_id(0)
    i = pl.program_id(1)
    h2 = mean_ref.shape[1]

    @pl.when((p == 0) & (i == 0))
    def _():
        xw1_s[...] = jnp.dot(
            x_ref[...], w1_ref[...], preferred_element_type=jnp.float32)

    @pl.when(p == 0)
    def _():
        temp = jnp.maximum(
            jnp.dot(adj_ref[...], xw1_s[...],
                    preferred_element_type=jnp.float32), 0.0)
        tw_s[pl.ds(i * ROW_BLK, ROW_BLK), :] = jnp.dot(
            temp, w23_ref[...], preferred_element_type=jnp.float32)

    @pl.when(p == 1)
    def _():
        mv = jnp.dot(adj_ref[...], tw_s[...],
                     preferred_element_type=jnp.float32)
        mean_ref[...] = mv[:, :h2]
        logvar_ref[...] = mv[:, h2:]


def _decoder_kernel(z_ref, zt_ref, out_ref):
    out_ref[...] = jnp.dot(z_ref[...], zt_ref[...],
                           preferred_element_type=jnp.float32)


def kernel(node_vectors, adj, W1, W2, W3):
    n, d = node_vectors.shape
    h1 = W1.shape[1]
    h2 = W2.shape[1]
    w23 = jnp.concatenate([W2, W3], axis=1)

    nblk = n // ROW_BLK
    last = nblk - 1

    mean, logvar = pl.pallas_call(
        _gcn_kernel,
        grid=(2, nblk),
        in_specs=[
            pl.BlockSpec((ROW_BLK, n), lambda p, i: (i, 0)),
            pl.BlockSpec((n, d), lambda p, i: (0, 0)),
            pl.BlockSpec((d, h1), lambda p, i: (0, 0)),
            pl.BlockSpec((h1, 2 * h2), lambda p, i: (0, 0)),
        ],
        out_specs=[
            pl.BlockSpec((ROW_BLK, h2),
                         lambda p, i: (jnp.where(p == 1, i, 0), 0)),
            pl.BlockSpec((ROW_BLK, h2),
                         lambda p, i: (jnp.where(p == 1, i, 0), 0)),
        ],
        out_shape=[
            jax.ShapeDtypeStruct((n, h2), jnp.float32),
            jax.ShapeDtypeStruct((n, h2), jnp.float32),
        ],
        scratch_shapes=[
            pltpu.VMEM((n, h1), jnp.float32),
            pltpu.VMEM((n, 2 * h2), jnp.float32),
        ],
        compiler_params=pltpu.CompilerParams(
            dimension_semantics=("arbitrary", "arbitrary")),
    )(adj, node_vectors, W1, w23)

    adj_dec = pl.pallas_call(
        _decoder_kernel,
        grid=(nblk,),
        in_specs=[
            pl.BlockSpec((ROW_BLK, h2), lambda i: (i, 0)),
            pl.BlockSpec((h2, n), lambda i: (0, 0)),
        ],
        out_specs=pl.BlockSpec((ROW_BLK, n), lambda i: (i, 0)),
        out_shape=jax.ShapeDtypeStruct((n, n), jnp.float32),
    )(mean, mean.T)

    return (adj_dec, mean, logvar)
